# Initial kernel scaffold; baseline (speedup 1.0000x reference)
import jax
import jax.numpy as jnp
from jax.experimental import pallas as pl


def kernel(x, edge_index, edge_attr, batch, demographics, emb, conv0_Wrel, conv0_brel, conv0_Wroot, conv1_Wrel, conv1_brel, conv1_Wroot, conv2_Wrel, conv2_brel, conv2_Wroot, gn0_alpha, gn0_gamma, gn0_beta, gn1_alpha, gn1_gamma, gn1_beta, demo_W, demo_b, cls_W1, cls_b1, bn1_g, bn1_b, cls_W2, cls_b2, bn2_g, bn2_b, cls_W3, cls_b3):
    raise NotImplementedError("write your pallas kernel here")


# speedup vs baseline: 1.0005x; 1.0005x over previous
import jax
import jax.numpy as jnp
from jax.experimental import pallas as pl

_N = 50000
_G = 128


def _head_body(gf_ref, demo_ref, demo_W_ref, demo_b_ref,
               W1_ref, b1_ref, g1_ref, be1_ref,
               W2_ref, b2_ref, g2_ref, be2_ref,
               W3_ref, b3_ref, out_ref):
    gf = gf_ref[...]
    df = jnp.dot(demo_ref[...], demo_W_ref[...],
                 preferred_element_type=jnp.float32) + demo_b_ref[...]
    comb = jnp.concatenate([gf, df], axis=1)
    z = jnp.dot(comb, W1_ref[...], preferred_element_type=jnp.float32) + b1_ref[...]
    z = (z / jnp.sqrt(1.0 + 1e-5)) * g1_ref[...] + be1_ref[...]
    z = jnp.where(z >= 0, z, 0.01 * z)
    z = jnp.dot(z, W2_ref[...], preferred_element_type=jnp.float32) + b2_ref[...]
    z = (z / jnp.sqrt(1.0 + 1e-5)) * g2_ref[...] + be2_ref[...]
    z = jnp.where(z >= 0, z, 0.01 * z)
    out_ref[...] = jnp.dot(z, W3_ref[...],
                           preferred_element_type=jnp.float32) + b3_ref[...]


def kernel(x, edge_index, edge_attr, batch, demographics, emb,
           conv0_Wrel, conv0_brel, conv0_Wroot,
           conv1_Wrel, conv1_brel, conv1_Wroot,
           conv2_Wrel, conv2_brel, conv2_Wroot,
           gn0_alpha, gn0_gamma, gn0_beta,
           gn1_alpha, gn1_gamma, gn1_beta,
           demo_W, demo_b,
           cls_W1, cls_b1, bn1_g, bn1_b,
           cls_W2, cls_b2, bn2_g, bn2_b,
           cls_W3, cls_b3):
    src = edge_index[0]
    dst = edge_index[1]
    ew = edge_attr[:, 0]
    counts = jnp.maximum(
        jax.ops.segment_sum(jnp.ones((_N,), jnp.float32), batch, num_segments=_G), 1.0)
    h = emb[x]
    convs = [(conv0_Wrel, conv0_brel, conv0_Wroot),
             (conv1_Wrel, conv1_brel, conv1_Wroot),
             (conv2_Wrel, conv2_brel, conv2_Wroot)]
    gns = [(gn0_alpha, gn0_gamma, gn0_beta), (gn1_alpha, gn1_gamma, gn1_beta)]
    for i in range(3):
        Wrel, brel, Wroot = convs[i]
        msg = h[src] * ew[:, None]
        agg = jax.ops.segment_sum(msg, dst, num_segments=_N)
        h = agg @ Wrel + brel + h @ Wroot
        if i < 2:
            alpha, gamma, beta = gns[i]
            mean = jax.ops.segment_sum(h, batch, num_segments=_G) / counts[:, None]
            sub = h - alpha * mean[batch]
            var = jax.ops.segment_sum(sub * sub, batch, num_segments=_G) / counts[:, None]
            h = gamma * sub / jnp.sqrt(var[batch] + 1e-5) + beta
            h = jnp.where(h >= 0, h, 0.01 * h)
    gf = jax.ops.segment_sum(h, batch, num_segments=_G) / counts[:, None]

    out = pl.pallas_call(
        _head_body,
        out_shape=jax.ShapeDtypeStruct((_G, 8), jnp.float32),
    )(gf, demographics, demo_W, demo_b,
      cls_W1, cls_b1, bn1_g, bn1_b,
      cls_W2, cls_b2, bn2_g, bn2_b,
      cls_W3, cls_b3)
    return out


# speedup vs baseline: 2.9401x; 2.9387x over previous
import functools

import jax
import jax.numpy as jnp
from jax import lax
from jax.experimental import pallas as pl
from jax.experimental.pallas import tpu as pltpu
from jax.experimental.pallas import tpu_sc as plsc

_N = 50000
_E = 800000
_G = 128
_NPAD = 50176
_EPAD = 802816
_BN = 1024
_NB = _NPAD // _BN
_NW = 32
_CH = 128
_CPH = 196
_SEGC = 28
_NSEG = _CPH // _SEGC

@functools.cache
def _mesh():
    return plsc.VectorSubcoreMesh(core_axis_name="c", subcore_axis_name="s",
                                  num_cores=2, num_subcores=16)


def _emb_gather_body(emb_hbm, x_hbm, out_hbm, xbuf, rows):
    c = lax.axis_index("c")
    s = lax.axis_index("s")
    w = c * 16 + s
    pltpu.sync_copy(x_hbm.at[w], xbuf)
    for t in range(14):
        pltpu.sync_copy(emb_hbm.at[xbuf.at[pl.ds(t * 112, 112)]],
                        rows.at[pl.ds(t * 112, 112)])
    pltpu.sync_copy(rows, out_hbm.at[pl.ds(w * 1568, 1568)])


def _make_emb_gather():
    return pl.kernel(
        _emb_gather_body,
        out_type=jax.ShapeDtypeStruct((_NPAD, 16), jnp.float32),
        mesh=_mesh(),
        compiler_params=pltpu.CompilerParams(use_tc_tiling_on_sc=False, needs_layout_passes=False),
        scratch_types=[
            pltpu.VMEM((1568,), jnp.int32),
            pltpu.VMEM((1568, 16), jnp.float32),
        ],
    )


def _make_edge_agg(di, halves, table_rows):
    rpt = _NPAD // 16

    def body(table_hbm, src_hbm, dst_hbm, ew_hbm, zeros_hbm, out_hbm,
             srcbuf, dstbuf, ewbuf, rows, aggsh):
        c = lax.axis_index("c")
        s = lax.axis_index("s")
        pltpu.sync_copy(zeros_hbm.at[pl.ds(s * rpt, rpt)],
                        aggsh.at[pl.ds(s * rpt, rpt)])
        plsc.subcore_barrier()
        iota16 = lax.iota(jnp.int32, 16)

        def segment(t, carry):
            h = t // _NSEG
            seg = t % _NSEG
            pltpu.sync_copy(
                src_hbm.at[c, s, h, pl.ds(seg * _SEGC * _CH, _SEGC * _CH)],
                srcbuf)
            pltpu.sync_copy(dst_hbm.at[c, s, h, pl.ds(seg * _SEGC, _SEGC)],
                            dstbuf)
            pltpu.sync_copy(
                ew_hbm.at[c, s, h, pl.ds(seg * _SEGC * _CH, _SEGC * _CH)],
                ewbuf)

            def chunk(j, carry2):
                pltpu.sync_copy(
                    table_hbm.at[srcbuf.at[pl.ds(j * _CH, _CH)]], rows)
                for k in range(_CH // 16):
                    ew16 = ewbuf[pl.ds(j * _CH + k * 16, 16)]
                    ri = iota16 + (k * 16)
                    for col in range(di):
                        ci = jnp.full((16,), col, jnp.int32)
                        v = plsc.load_gather(rows, [ri, ci])
                        plsc.store_scatter(rows, [ri, ci], v * ew16)
                pltpu.sync_copy(rows, aggsh.at[dstbuf.at[j]], add=True)
                return carry2

            lax.fori_loop(0, _SEGC, chunk, 0)
            return carry

        lax.fori_loop(0, halves * _NSEG, segment, 0)
        plsc.subcore_barrier()
        pltpu.sync_copy(aggsh.at[pl.ds(s * rpt, rpt)],
                        out_hbm.at[c, pl.ds(s * rpt, rpt)])

    return pl.kernel(
        body,
        out_type=jax.ShapeDtypeStruct((2, _NPAD, di), jnp.float32),
        mesh=_mesh(),
        compiler_params=pltpu.CompilerParams(use_tc_tiling_on_sc=False, needs_layout_passes=False),
        scratch_types=[
            pltpu.VMEM((_SEGC * _CH,), jnp.int32),
            pltpu.VMEM((_SEGC, _CH), jnp.int32),
            pltpu.VMEM((_SEGC * _CH,), jnp.float32),
            pltpu.VMEM((_CH, di), jnp.float32),
            pltpu.MemorySpace.VMEM_SHARED((_NPAD, di), jnp.float32),
        ],
    )


def _onehot(bt):
    g = lax.broadcasted_iota(jnp.int32, (bt.shape[0], _G), 1)
    return (bt[:, None] == g).astype(jnp.float32)


def _convA_body(aggp_ref, h_ref, wrel_ref, brel_ref, wroot_ref, batch_ref,
                hraw_ref, stats_ref):
    b = pl.program_id(0)
    agg = aggp_ref[0] + aggp_ref[1]
    z = (jnp.dot(agg, wrel_ref[...], preferred_element_type=jnp.float32)
         + brel_ref[...]
         + jnp.dot(h_ref[...], wroot_ref[...],
                   preferred_element_type=jnp.float32))
    hraw_ref[...] = z
    P = _onehot(batch_ref[0, 0])
    dn = (((0,), (0,)), ((), ()))
    s1 = lax.dot_general(P, z, dn, preferred_element_type=jnp.float32)
    s2 = lax.dot_general(P, z * z, dn, preferred_element_type=jnp.float32)
    s0 = jnp.broadcast_to(jnp.sum(P, axis=0)[:, None], s1.shape)

    @pl.when(b == 0)
    def _():
        stats_ref[...] = jnp.zeros_like(stats_ref)

    stats_ref[0] += s0
    stats_ref[1] += s1
    stats_ref[2] += s2


def _make_convA(di, do):
    return pl.pallas_call(
        _convA_body,
        grid=(_NB,),
        in_specs=[
            pl.BlockSpec((2, _BN, di), lambda b: (0, b, 0)),
            pl.BlockSpec((_BN, di), lambda b: (b, 0)),
            pl.BlockSpec((di, do), lambda b: (0, 0)),
            pl.BlockSpec((1, do), lambda b: (0, 0)),
            pl.BlockSpec((di, do), lambda b: (0, 0)),
            pl.BlockSpec((1, 1, _BN), lambda b: (b, 0, 0)),
        ],
        out_specs=[
            pl.BlockSpec((_BN, do), lambda b: (b, 0)),
            pl.BlockSpec((3, _G, do), lambda b: (0, 0, 0)),
        ],
        out_shape=[
            jax.ShapeDtypeStruct((_NPAD, do), jnp.float32),
            jax.ShapeDtypeStruct((3, _G, do), jnp.float32),
        ],
    )


def _gn_scale_shift(stats_ref, alpha, gamma, beta):
    cnt = jnp.maximum(stats_ref[0], 1.0)
    mean = stats_ref[1] / cnt
    ex2 = stats_ref[2] / cnt
    var = ex2 - (2.0 * alpha - alpha * alpha) * mean * mean
    scale = gamma / jnp.sqrt(var + 1e-5)
    shift = beta - alpha * mean * scale
    return scale, shift


def _gn0_body(hraw_ref, stats_ref, batch_ref, a_ref, g_ref, be_ref, out_ref):
    scale, shift = _gn_scale_shift(stats_ref, a_ref[...], g_ref[...],
                                   be_ref[...])
    P = _onehot(batch_ref[0, 0])
    z = (hraw_ref[...]
         * jnp.dot(P, scale, preferred_element_type=jnp.float32)
         + jnp.dot(P, shift, preferred_element_type=jnp.float32))
    out_ref[...] = jnp.where(z >= 0, z, 0.01 * z)


def _make_gn0(do):
    return pl.pallas_call(
        _gn0_body,
        grid=(_NB,),
        in_specs=[
            pl.BlockSpec((_BN, do), lambda b: (b, 0)),
            pl.BlockSpec((3, _G, do), lambda b: (0, 0, 0)),
            pl.BlockSpec((1, 1, _BN), lambda b: (b, 0, 0)),
            pl.BlockSpec((1, do), lambda b: (0, 0)),
            pl.BlockSpec((1, do), lambda b: (0, 0)),
            pl.BlockSpec((1, do), lambda b: (0, 0)),
        ],
        out_specs=pl.BlockSpec((_BN, do), lambda b: (b, 0)),
        out_shape=jax.ShapeDtypeStruct((_NPAD, do), jnp.float32),
    )


def _gn1_body(hraw_ref, stats_ref, batch_ref, a_ref, g_ref, be_ref,
              split_ref, sh_ref):
    b = pl.program_id(0)
    scale, shift = _gn_scale_shift(stats_ref, a_ref[...], g_ref[...],
                                   be_ref[...])
    P = _onehot(batch_ref[0, 0])
    z = (hraw_ref[...]
         * jnp.dot(P, scale, preferred_element_type=jnp.float32)
         + jnp.dot(P, shift, preferred_element_type=jnp.float32))
    hh = jnp.where(z >= 0, z, 0.01 * z)
    split_ref[0] = hh[:, :32]
    split_ref[1] = hh[:, 32:]
    dn = (((0,), (0,)), ((), ()))
    s = lax.dot_general(P, hh, dn, preferred_element_type=jnp.float32)

    @pl.when(b == 0)
    def _():
        sh_ref[...] = jnp.zeros_like(sh_ref)

    sh_ref[...] += s


def _make_gn1():
    return pl.pallas_call(
        _gn1_body,
        grid=(_NB,),
        in_specs=[
            pl.BlockSpec((_BN, 64), lambda b: (b, 0)),
            pl.BlockSpec((3, _G, 64), lambda b: (0, 0, 0)),
            pl.BlockSpec((1, 1, _BN), lambda b: (b, 0, 0)),
            pl.BlockSpec((1, 64), lambda b: (0, 0)),
            pl.BlockSpec((1, 64), lambda b: (0, 0)),
            pl.BlockSpec((1, 64), lambda b: (0, 0)),
        ],
        out_specs=[
            pl.BlockSpec((2, _BN, 32), lambda b: (0, b, 0)),
            pl.BlockSpec((_G, 64), lambda b: (0, 0)),
        ],
        out_shape=[
            jax.ShapeDtypeStruct((2, _NPAD, 32), jnp.float32),
            jax.ShapeDtypeStruct((_G, 64), jnp.float32),
        ],
    )


def _pool_body(agg_ref, batch_ref, sa_ref):
    b = pl.program_id(0)
    P = _onehot(batch_ref[0, 0])
    dn = (((0,), (0,)), ((), ()))

    @pl.when(b == 0)
    def _():
        sa_ref[...] = jnp.zeros_like(sa_ref)

    sa_ref[0] += lax.dot_general(P, agg_ref[0], dn,
                                 preferred_element_type=jnp.float32)
    sa_ref[1] += lax.dot_general(P, agg_ref[1], dn,
                                 preferred_element_type=jnp.float32)


def _make_pool():
    return pl.pallas_call(
        _pool_body,
        grid=(_NB,),
        in_specs=[
            pl.BlockSpec((2, _BN, 32), lambda b: (0, b, 0)),
            pl.BlockSpec((1, 1, _BN), lambda b: (b, 0, 0)),
        ],
        out_specs=pl.BlockSpec((2, _G, 32), lambda b: (0, 0, 0)),
        out_shape=jax.ShapeDtypeStruct((2, _G, 32), jnp.float32),
    )


def _head_body(sa_ref, sh_ref, stats0_ref, demo_ref,
               wrel_ref, brel_ref, wroot_ref,
               demo_W_ref, demo_b_ref,
               W1_ref, b1_ref, g1_ref, be1_ref,
               W2_ref, b2_ref, g2_ref, be2_ref,
               W3_ref, b3_ref, out_ref):
    aggseg = jnp.concatenate([sa_ref[0], sa_ref[1]], axis=1)
    cnt_raw = stats0_ref[0][:, 0:1]
    cnt = jnp.maximum(cnt_raw, 1.0)
    gf = (jnp.dot(aggseg, wrel_ref[...], preferred_element_type=jnp.float32)
          + jnp.dot(sh_ref[...], wroot_ref[...],
                    preferred_element_type=jnp.float32)
          + cnt_raw * brel_ref[...]) / cnt
    df = jnp.dot(demo_ref[...], demo_W_ref[...],
                 preferred_element_type=jnp.float32) + demo_b_ref[...]
    comb = jnp.concatenate([gf, df], axis=1)
    z = jnp.dot(comb, W1_ref[...], preferred_element_type=jnp.float32) + b1_ref[...]
    z = (z / jnp.sqrt(1.0 + 1e-5)) * g1_ref[...] + be1_ref[...]
    z = jnp.where(z >= 0, z, 0.01 * z)
    z = jnp.dot(z, W2_ref[...], preferred_element_type=jnp.float32) + b2_ref[...]
    z = (z / jnp.sqrt(1.0 + 1e-5)) * g2_ref[...] + be2_ref[...]
    z = jnp.where(z >= 0, z, 0.01 * z)
    out_ref[...] = jnp.dot(z, W3_ref[...],
                           preferred_element_type=jnp.float32) + b3_ref[...]


def _make_head():
    return pl.pallas_call(
        _head_body,
        out_shape=jax.ShapeDtypeStruct((_G, 8), jnp.float32),
    )


def kernel(x, edge_index, edge_attr, batch, demographics, emb,
           conv0_Wrel, conv0_brel, conv0_Wroot,
           conv1_Wrel, conv1_brel, conv1_Wroot,
           conv2_Wrel, conv2_brel, conv2_Wroot,
           gn0_alpha, gn0_gamma, gn0_beta,
           gn1_alpha, gn1_gamma, gn1_beta,
           demo_W, demo_b,
           cls_W1, cls_b1, bn1_g, bn1_b,
           cls_W2, cls_b2, bn2_g, bn2_b,
           cls_W3, cls_b3):
    src = edge_index[0]
    dst = edge_index[1]
    ew = edge_attr[:, 0]

    epad = _EPAD - _E
    fill = jnp.arange(epad, dtype=jnp.int32) % _N
    src_p = jnp.concatenate([src, fill])
    dst_p = jnp.concatenate([dst, fill])
    ew_p = jnp.concatenate([ew, jnp.zeros((epad,), jnp.float32)])

    src_a1 = src_p.reshape(2, 16, 1, _CPH * _CH)
    dst_a1 = dst_p.reshape(2, 16, 1, _CPH, _CH)
    ew_a1 = ew_p.reshape(2, 16, 1, _CPH * _CH)

    src_a2 = jnp.stack([src_p, src_p + _NPAD]).reshape(2, 16, 2, _CPH * _CH)
    dst_a2 = jnp.broadcast_to(dst_p, (2, _EPAD)).reshape(2, 16, 2, _CPH, _CH)
    ew_a2 = jnp.broadcast_to(ew_p, (2, _EPAD)).reshape(2, 16, 2, _CPH * _CH)

    x_a = jnp.concatenate(
        [x, jnp.zeros((_NPAD - _N,), jnp.int32)]).reshape(_NW, 1568)
    batch3 = jnp.concatenate(
        [batch, jnp.full((_NPAD - _N,), _G, jnp.int32)]).reshape(_NB, 1, _BN)

    z16 = jnp.zeros((_NPAD, 16), jnp.float32)
    z32 = jnp.zeros((_NPAD, 32), jnp.float32)

    r1 = lambda v: v.reshape(1, -1)

    h0 = _make_emb_gather()(emb, x_a)
    agg0p = _make_edge_agg(16, 1, _NPAD)(h0, src_a1, dst_a1, ew_a1, z16)
    hraw0, stats0 = _make_convA(16, 32)(
        agg0p, h0, conv0_Wrel, r1(conv0_brel), conv0_Wroot, batch3)
    h1 = _make_gn0(32)(hraw0, stats0, batch3,
                       r1(gn0_alpha), r1(gn0_gamma), r1(gn0_beta))

    agg1p = _make_edge_agg(32, 1, _NPAD)(h1, src_a1, dst_a1, ew_a1, z32)
    hraw1, stats1 = _make_convA(32, 64)(
        agg1p, h1, conv1_Wrel, r1(conv1_brel), conv1_Wroot, batch3)
    h2split, sh2 = _make_gn1()(hraw1, stats1, batch3,
                               r1(gn1_alpha), r1(gn1_gamma), r1(gn1_beta))

    h2flat = h2split.reshape(2 * _NPAD, 32)
    agg2 = _make_edge_agg(32, 2, 2 * _NPAD)(
        h2flat, src_a2, dst_a2, ew_a2, z32)
    sa = _make_pool()(agg2, batch3)

    out = _make_head()(
        sa, sh2, stats0, demographics,
        conv2_Wrel, r1(conv2_brel), conv2_Wroot,
        demo_W, r1(demo_b),
        cls_W1, r1(cls_b1), r1(bn1_g), r1(bn1_b),
        cls_W2, r1(cls_b2), r1(bn2_g), r1(bn2_b),
        cls_W3, r1(cls_b3))
    return out


# speedup vs baseline: 3.4589x; 1.1765x over previous
import functools

import jax
import jax.numpy as jnp
from jax import lax
from jax.experimental import pallas as pl
from jax.experimental.pallas import tpu as pltpu
from jax.experimental.pallas import tpu_sc as plsc

_N = 50000
_E = 800000
_G = 128
_NPAD = 50176
_EPAD = 802816
_BN = 1024
_NB = _NPAD // _BN
_NW = 32
_CH = 128
_CPH = 196
_SEGC = 28
_NSEG = _CPH // _SEGC

@functools.cache
def _mesh():
    return plsc.VectorSubcoreMesh(core_axis_name="c", subcore_axis_name="s",
                                  num_cores=2, num_subcores=16)


def _emb_gather_body(emb_hbm, x_hbm, out_hbm, xbuf, rows):
    c = lax.axis_index("c")
    s = lax.axis_index("s")
    w = c * 16 + s
    pltpu.sync_copy(x_hbm.at[w], xbuf)
    for t in range(14):
        pltpu.sync_copy(emb_hbm.at[xbuf.at[pl.ds(t * 112, 112)]],
                        rows.at[pl.ds(t * 112, 112)])
    pltpu.sync_copy(rows, out_hbm.at[pl.ds(w * 1568, 1568)])


def _make_emb_gather():
    return pl.kernel(
        _emb_gather_body,
        out_type=jax.ShapeDtypeStruct((_NPAD, 16), jnp.float32),
        mesh=_mesh(),
        compiler_params=pltpu.CompilerParams(use_tc_tiling_on_sc=False, needs_layout_passes=False),
        scratch_types=[
            pltpu.VMEM((1568,), jnp.int32),
            pltpu.VMEM((1568, 16), jnp.float32),
        ],
    )


def _make_edge_agg(di, halves, table_rows):
    rpt = _NPAD // 16

    def body(table_hbm, src_hbm, dst_hbm, ew_hbm, zeros_hbm, out_hbm,
             srcbuf, dstbuf, ewbuf, rows0, rows1, srows0, srows1,
             sg0, sg1, ss0, ss1, aggsh):
        c = lax.axis_index("c")
        s = lax.axis_index("s")
        rows = (rows0, rows1)
        srows = (srows0, srows1)
        sg = (sg0, sg1)
        ss = (ss0, ss1)
        pltpu.sync_copy(zeros_hbm.at[pl.ds(s * rpt, rpt)],
                        aggsh.at[pl.ds(s * rpt, rpt)])
        plsc.subcore_barrier()
        iota16 = lax.iota(jnp.int32, 16)

        def g_desc(j, b):
            return pltpu.make_async_copy(
                table_hbm.at[srcbuf.at[pl.ds(j * _CH, _CH)]], rows[b], sg[b])

        def s_start(j, b):
            pltpu.async_copy(srows[b], aggsh.at[dstbuf.at[j]], ss[b],
                             add=True)

        def s_wait(j, b):
            pltpu.make_async_copy(srows[b], aggsh.at[dstbuf.at[j]],
                                  ss[b]).wait()

        def scale(b, j):
            for k in range(_CH // 16):
                ew16 = ewbuf[pl.ds(j * _CH + k * 16, 16)]
                ri = iota16 + (k * 16)
                for col in range(di):
                    ci = jnp.full((16,), col, jnp.int32)
                    v = plsc.load_gather(rows[b], [ri, ci])
                    plsc.store_scatter(srows[b], [ri, ci], v * ew16)

        def segment(t, carry):
            h = t // _NSEG
            seg = t % _NSEG
            pltpu.sync_copy(
                src_hbm.at[c, s, h, pl.ds(seg * _SEGC * _CH, _SEGC * _CH)],
                srcbuf)
            pltpu.sync_copy(dst_hbm.at[c, s, h, pl.ds(seg * _SEGC, _SEGC)],
                            dstbuf)
            pltpu.sync_copy(
                ew_hbm.at[c, s, h, pl.ds(seg * _SEGC * _CH, _SEGC * _CH)],
                ewbuf)
            for b in range(2):
                g_desc(b, b).start()

            def group(g, carry2):
                for b in range(2):
                    j = g * 2 + b
                    g_desc(j, b).wait()

                    @pl.when(g > 0)
                    def _():
                        s_wait(j, b)

                    scale(b, j)

                    @pl.when(j + 2 < _SEGC)
                    def _():
                        g_desc(j + 2, b).start()

                    s_start(j, b)
                return carry2

            lax.fori_loop(0, _SEGC // 2, group, 0)
            for b in range(2):
                s_wait(b, b)
            return carry

        lax.fori_loop(0, halves * _NSEG, segment, 0)
        plsc.subcore_barrier()
        pltpu.sync_copy(aggsh.at[pl.ds(s * rpt, rpt)],
                        out_hbm.at[c, pl.ds(s * rpt, rpt)])

    return pl.kernel(
        body,
        out_type=jax.ShapeDtypeStruct((2, _NPAD, di), jnp.float32),
        mesh=_mesh(),
        compiler_params=pltpu.CompilerParams(use_tc_tiling_on_sc=False, needs_layout_passes=False),
        scratch_types=[
            pltpu.VMEM((_SEGC * _CH,), jnp.int32),
            pltpu.VMEM((_SEGC, _CH), jnp.int32),
            pltpu.VMEM((_SEGC * _CH,), jnp.float32),
            pltpu.VMEM((_CH, di), jnp.float32),
            pltpu.VMEM((_CH, di), jnp.float32),
            pltpu.VMEM((_CH, di), jnp.float32),
            pltpu.VMEM((_CH, di), jnp.float32),
            pltpu.SemaphoreType.DMA,
            pltpu.SemaphoreType.DMA,
            pltpu.SemaphoreType.DMA,
            pltpu.SemaphoreType.DMA,
            pltpu.MemorySpace.VMEM_SHARED((_NPAD, di), jnp.float32),
        ],
    )


def _onehot(bt):
    g = lax.broadcasted_iota(jnp.int32, (bt.shape[0], _G), 1)
    return (bt[:, None] == g).astype(jnp.float32)


def _convA_body(aggp_ref, h_ref, wrel_ref, brel_ref, wroot_ref, batch_ref,
                hraw_ref, stats_ref):
    b = pl.program_id(0)
    agg = aggp_ref[0] + aggp_ref[1]
    z = (jnp.dot(agg, wrel_ref[...], preferred_element_type=jnp.float32)
         + brel_ref[...]
         + jnp.dot(h_ref[...], wroot_ref[...],
                   preferred_element_type=jnp.float32))
    hraw_ref[...] = z
    P = _onehot(batch_ref[0, 0])
    dn = (((0,), (0,)), ((), ()))
    s1 = lax.dot_general(P, z, dn, preferred_element_type=jnp.float32)
    s2 = lax.dot_general(P, z * z, dn, preferred_element_type=jnp.float32)
    s0 = jnp.broadcast_to(jnp.sum(P, axis=0)[:, None], s1.shape)

    @pl.when(b == 0)
    def _():
        stats_ref[...] = jnp.zeros_like(stats_ref)

    stats_ref[0] += s0
    stats_ref[1] += s1
    stats_ref[2] += s2


def _make_convA(di, do):
    return pl.pallas_call(
        _convA_body,
        grid=(_NB,),
        in_specs=[
            pl.BlockSpec((2, _BN, di), lambda b: (0, b, 0)),
            pl.BlockSpec((_BN, di), lambda b: (b, 0)),
            pl.BlockSpec((di, do), lambda b: (0, 0)),
            pl.BlockSpec((1, do), lambda b: (0, 0)),
            pl.BlockSpec((di, do), lambda b: (0, 0)),
            pl.BlockSpec((1, 1, _BN), lambda b: (b, 0, 0)),
        ],
        out_specs=[
            pl.BlockSpec((_BN, do), lambda b: (b, 0)),
            pl.BlockSpec((3, _G, do), lambda b: (0, 0, 0)),
        ],
        out_shape=[
            jax.ShapeDtypeStruct((_NPAD, do), jnp.float32),
            jax.ShapeDtypeStruct((3, _G, do), jnp.float32),
        ],
    )


def _gn_scale_shift(stats_ref, alpha, gamma, beta):
    cnt = jnp.maximum(stats_ref[0], 1.0)
    mean = stats_ref[1] / cnt
    ex2 = stats_ref[2] / cnt
    var = ex2 - (2.0 * alpha - alpha * alpha) * mean * mean
    scale = gamma / jnp.sqrt(var + 1e-5)
    shift = beta - alpha * mean * scale
    return scale, shift


def _gn0_body(hraw_ref, stats_ref, batch_ref, a_ref, g_ref, be_ref, out_ref):
    scale, shift = _gn_scale_shift(stats_ref, a_ref[...], g_ref[...],
                                   be_ref[...])
    P = _onehot(batch_ref[0, 0])
    z = (hraw_ref[...]
         * jnp.dot(P, scale, preferred_element_type=jnp.float32)
         + jnp.dot(P, shift, preferred_element_type=jnp.float32))
    out_ref[...] = jnp.where(z >= 0, z, 0.01 * z)


def _make_gn0(do):
    return pl.pallas_call(
        _gn0_body,
        grid=(_NB,),
        in_specs=[
            pl.BlockSpec((_BN, do), lambda b: (b, 0)),
            pl.BlockSpec((3, _G, do), lambda b: (0, 0, 0)),
            pl.BlockSpec((1, 1, _BN), lambda b: (b, 0, 0)),
            pl.BlockSpec((1, do), lambda b: (0, 0)),
            pl.BlockSpec((1, do), lambda b: (0, 0)),
            pl.BlockSpec((1, do), lambda b: (0, 0)),
        ],
        out_specs=pl.BlockSpec((_BN, do), lambda b: (b, 0)),
        out_shape=jax.ShapeDtypeStruct((_NPAD, do), jnp.float32),
    )


def _gn1_body(hraw_ref, stats_ref, batch_ref, a_ref, g_ref, be_ref,
              split_ref, sh_ref):
    b = pl.program_id(0)
    scale, shift = _gn_scale_shift(stats_ref, a_ref[...], g_ref[...],
                                   be_ref[...])
    P = _onehot(batch_ref[0, 0])
    z = (hraw_ref[...]
         * jnp.dot(P, scale, preferred_element_type=jnp.float32)
         + jnp.dot(P, shift, preferred_element_type=jnp.float32))
    hh = jnp.where(z >= 0, z, 0.01 * z)
    split_ref[0] = hh[:, :32]
    split_ref[1] = hh[:, 32:]
    dn = (((0,), (0,)), ((), ()))
    s = lax.dot_general(P, hh, dn, preferred_element_type=jnp.float32)

    @pl.when(b == 0)
    def _():
        sh_ref[...] = jnp.zeros_like(sh_ref)

    sh_ref[...] += s


def _make_gn1():
    return pl.pallas_call(
        _gn1_body,
        grid=(_NB,),
        in_specs=[
            pl.BlockSpec((_BN, 64), lambda b: (b, 0)),
            pl.BlockSpec((3, _G, 64), lambda b: (0, 0, 0)),
            pl.BlockSpec((1, 1, _BN), lambda b: (b, 0, 0)),
            pl.BlockSpec((1, 64), lambda b: (0, 0)),
            pl.BlockSpec((1, 64), lambda b: (0, 0)),
            pl.BlockSpec((1, 64), lambda b: (0, 0)),
        ],
        out_specs=[
            pl.BlockSpec((2, _BN, 32), lambda b: (0, b, 0)),
            pl.BlockSpec((_G, 64), lambda b: (0, 0)),
        ],
        out_shape=[
            jax.ShapeDtypeStruct((2, _NPAD, 32), jnp.float32),
            jax.ShapeDtypeStruct((_G, 64), jnp.float32),
        ],
    )


def _pool_body(agg_ref, batch_ref, sa_ref):
    b = pl.program_id(0)
    P = _onehot(batch_ref[0, 0])
    dn = (((0,), (0,)), ((), ()))

    @pl.when(b == 0)
    def _():
        sa_ref[...] = jnp.zeros_like(sa_ref)

    sa_ref[0] += lax.dot_general(P, agg_ref[0], dn,
                                 preferred_element_type=jnp.float32)
    sa_ref[1] += lax.dot_general(P, agg_ref[1], dn,
                                 preferred_element_type=jnp.float32)


def _make_pool():
    return pl.pallas_call(
        _pool_body,
        grid=(_NB,),
        in_specs=[
            pl.BlockSpec((2, _BN, 32), lambda b: (0, b, 0)),
            pl.BlockSpec((1, 1, _BN), lambda b: (b, 0, 0)),
        ],
        out_specs=pl.BlockSpec((2, _G, 32), lambda b: (0, 0, 0)),
        out_shape=jax.ShapeDtypeStruct((2, _G, 32), jnp.float32),
    )


def _head_body(sa_ref, sh_ref, stats0_ref, demo_ref,
               wrel_ref, brel_ref, wroot_ref,
               demo_W_ref, demo_b_ref,
               W1_ref, b1_ref, g1_ref, be1_ref,
               W2_ref, b2_ref, g2_ref, be2_ref,
               W3_ref, b3_ref, out_ref):
    aggseg = jnp.concatenate([sa_ref[0], sa_ref[1]], axis=1)
    cnt_raw = stats0_ref[0][:, 0:1]
    cnt = jnp.maximum(cnt_raw, 1.0)
    gf = (jnp.dot(aggseg, wrel_ref[...], preferred_element_type=jnp.float32)
          + jnp.dot(sh_ref[...], wroot_ref[...],
                    preferred_element_type=jnp.float32)
          + cnt_raw * brel_ref[...]) / cnt
    df = jnp.dot(demo_ref[...], demo_W_ref[...],
                 preferred_element_type=jnp.float32) + demo_b_ref[...]
    comb = jnp.concatenate([gf, df], axis=1)
    z = jnp.dot(comb, W1_ref[...], preferred_element_type=jnp.float32) + b1_ref[...]
    z = (z / jnp.sqrt(1.0 + 1e-5)) * g1_ref[...] + be1_ref[...]
    z = jnp.where(z >= 0, z, 0.01 * z)
    z = jnp.dot(z, W2_ref[...], preferred_element_type=jnp.float32) + b2_ref[...]
    z = (z / jnp.sqrt(1.0 + 1e-5)) * g2_ref[...] + be2_ref[...]
    z = jnp.where(z >= 0, z, 0.01 * z)
    out_ref[...] = jnp.dot(z, W3_ref[...],
                           preferred_element_type=jnp.float32) + b3_ref[...]


def _make_head():
    return pl.pallas_call(
        _head_body,
        out_shape=jax.ShapeDtypeStruct((_G, 8), jnp.float32),
    )


def kernel(x, edge_index, edge_attr, batch, demographics, emb,
           conv0_Wrel, conv0_brel, conv0_Wroot,
           conv1_Wrel, conv1_brel, conv1_Wroot,
           conv2_Wrel, conv2_brel, conv2_Wroot,
           gn0_alpha, gn0_gamma, gn0_beta,
           gn1_alpha, gn1_gamma, gn1_beta,
           demo_W, demo_b,
           cls_W1, cls_b1, bn1_g, bn1_b,
           cls_W2, cls_b2, bn2_g, bn2_b,
           cls_W3, cls_b3):
    src = edge_index[0]
    dst = edge_index[1]
    ew = edge_attr[:, 0]

    epad = _EPAD - _E
    fill = jnp.arange(epad, dtype=jnp.int32) % _N
    src_p = jnp.concatenate([src, fill])
    dst_p = jnp.concatenate([dst, fill])
    ew_p = jnp.concatenate([ew, jnp.zeros((epad,), jnp.float32)])

    src_a1 = src_p.reshape(2, 16, 1, _CPH * _CH)
    dst_a1 = dst_p.reshape(2, 16, 1, _CPH, _CH)
    ew_a1 = ew_p.reshape(2, 16, 1, _CPH * _CH)

    src_a2 = jnp.stack([src_p, src_p + _NPAD]).reshape(2, 16, 2, _CPH * _CH)
    dst_a2 = jnp.broadcast_to(dst_p, (2, _EPAD)).reshape(2, 16, 2, _CPH, _CH)
    ew_a2 = jnp.broadcast_to(ew_p, (2, _EPAD)).reshape(2, 16, 2, _CPH * _CH)

    x_a = jnp.concatenate(
        [x, jnp.zeros((_NPAD - _N,), jnp.int32)]).reshape(_NW, 1568)
    batch3 = jnp.concatenate(
        [batch, jnp.full((_NPAD - _N,), _G, jnp.int32)]).reshape(_NB, 1, _BN)

    z16 = jnp.zeros((_NPAD, 16), jnp.float32)
    z32 = jnp.zeros((_NPAD, 32), jnp.float32)

    r1 = lambda v: v.reshape(1, -1)

    h0 = _make_emb_gather()(emb, x_a)
    agg0p = _make_edge_agg(16, 1, _NPAD)(h0, src_a1, dst_a1, ew_a1, z16)
    hraw0, stats0 = _make_convA(16, 32)(
        agg0p, h0, conv0_Wrel, r1(conv0_brel), conv0_Wroot, batch3)
    h1 = _make_gn0(32)(hraw0, stats0, batch3,
                       r1(gn0_alpha), r1(gn0_gamma), r1(gn0_beta))

    agg1p = _make_edge_agg(32, 1, _NPAD)(h1, src_a1, dst_a1, ew_a1, z32)
    hraw1, stats1 = _make_convA(32, 64)(
        agg1p, h1, conv1_Wrel, r1(conv1_brel), conv1_Wroot, batch3)
    h2split, sh2 = _make_gn1()(hraw1, stats1, batch3,
                               r1(gn1_alpha), r1(gn1_gamma), r1(gn1_beta))

    h2flat = h2split.reshape(2 * _NPAD, 32)
    agg2 = _make_edge_agg(32, 2, 2 * _NPAD)(
        h2flat, src_a2, dst_a2, ew_a2, z32)
    sa = _make_pool()(agg2, batch3)

    out = _make_head()(
        sa, sh2, stats0, demographics,
        conv2_Wrel, r1(conv2_brel), conv2_Wroot,
        demo_W, r1(demo_b),
        cls_W1, r1(cls_b1), r1(bn1_g), r1(bn1_b),
        cls_W2, r1(cls_b2), r1(bn2_g), r1(bn2_b),
        cls_W3, r1(cls_b3))
    return out
